# baseline (speedup 1.0000x reference)
import functools

import jax
import jax.numpy as jnp
from jax.experimental import pallas as pl
from jax.experimental.pallas import tpu as pltpu

_B, _S, _D = 2, 2048, 768
_G, _FPG, _DFE, _DH, _DRH = 8, 8, 64, 64, 64
_GH = _G * _DH


def _gelu(x):
    return 0.5 * x * (1.0 + jax.lax.erf(x * 0.7071067811865476))


def _moe_body(x_ref, f_ref, lng_ref, lnb_ref, wr1_ref, wg_ref, we1_ref,
              wr2_ref, bg_ref, br1_ref, be1_ref, br2_ref, we2_ref, be2_ref,
              out_ref, wh_s, wgbd_s, wr1e_s, wr2bd_s, spread_s):
    @pl.when(pl.program_id(0) == 0)
    def _init():
        wgbd_s[...] = jnp.zeros_like(wgbd_s)
        wr1e_s[...] = jnp.zeros_like(wr1e_s)
        wr2bd_s[...] = jnp.zeros_like(wr2bd_s)
        for g in range(_G):
            wh_s[:, g * _DRH:(g + 1) * _DRH] = wr1_ref[g, :_D, :]
            wh_s[:, _GH + g * _DH:_GH + (g + 1) * _DH] = we1_ref[g]
            wgbd_s[g * _FPG:(g + 1) * _FPG, g * _DFE:(g + 1) * _DFE] = (
                wg_ref[g * _FPG:(g + 1) * _FPG, :])
            wr1e_s[g * _DFE:(g + 1) * _DFE, g * _DRH:(g + 1) * _DRH] = (
                wr1_ref[g, _D:, :])
            wr2bd_s[g * _DRH:(g + 1) * _DRH, g:g + 1] = (
                wr2_ref[g * _DRH:(g + 1) * _DRH, :])
        r8 = jax.lax.broadcasted_iota(jnp.int32, (_G, _GH), 0)
        c512 = jax.lax.broadcasted_iota(jnp.int32, (_G, _GH), 1)
        spread_s[...] = (c512 // _DH == r8).astype(jnp.float32)

    x = x_ref[...]
    mu = jnp.mean(x, axis=1, keepdims=True)
    xc = x - mu
    var = jnp.mean(xc * xc, axis=1, keepdims=True)
    h = xc * jax.lax.rsqrt(var + 1e-5) * lng_ref[...] + lnb_ref[...]

    dot = functools.partial(jnp.dot, preferred_element_type=jnp.float32)
    hw = dot(h, wh_s[...])
    emb = dot(f_ref[...], wgbd_s[...]) + bg_ref[...]
    r1 = _gelu(hw[:, :_GH] + dot(emb, wr1e_s[...]) + br1_ref[...])
    e1 = _gelu(hw[:, _GH:] + be1_ref[...])

    logits = dot(r1, wr2bd_s[...]) + br2_ref[...]
    m1 = jnp.max(logits, axis=1, keepdims=True)
    l2 = jnp.where(logits == m1, -jnp.inf, logits)
    m2 = jnp.max(l2, axis=1, keepdims=True)
    inv = 1.0 / (1.0 + jnp.exp(m2 - m1))
    gw = jnp.where(logits >= m2, jnp.exp(logits - m1), 0.0) * inv

    e1w = e1 * dot(gw, spread_s[...])
    out_ref[...] = dot(e1w, we2_ref[...]) + dot(gw, be2_ref[...])


def kernel(hidden, features, ln_g, ln_b, Wg, bg, Wr1, br1, Wr2, br2,
           We1, be1, We2, be2):
    n = _B * _S
    x2 = hidden.reshape(n, _D)
    f2 = features.reshape(n, _G * _FPG)

    wg2 = Wg.reshape(_G * _FPG, _DFE)
    wr2r = Wr2.reshape(_GH, 1)
    we2c = We2.reshape(_GH, _D)
    lng2 = ln_g.reshape(1, _D)
    lnb2 = ln_b.reshape(1, _D)
    bgf = bg.reshape(1, _G * _DFE)
    br1f = br1.reshape(1, _G * _DRH)
    be1f = be1.reshape(1, _GH)
    br2f = br2.reshape(1, _G)

    tblk = 512
    grid = (n // tblk,)
    row = lambda i: (i, 0)

    def wspec(a):
        return pl.BlockSpec(a.shape, lambda *_: (0,) * a.ndim)

    out = pl.pallas_call(
        _moe_body,
        grid=grid,
        in_specs=[
            pl.BlockSpec((tblk, _D), row),
            pl.BlockSpec((tblk, _G * _FPG), row),
            wspec(lng2), wspec(lnb2), wspec(Wr1), wspec(wg2), wspec(We1),
            wspec(wr2r), wspec(bgf), wspec(br1f), wspec(be1f), wspec(br2f),
            wspec(we2c), wspec(be2),
        ],
        out_specs=pl.BlockSpec((tblk, _D), row),
        out_shape=jax.ShapeDtypeStruct((n, _D), jnp.float32),
        scratch_shapes=[
            pltpu.VMEM((_D, 2 * _GH), jnp.float32),
            pltpu.VMEM((_G * _FPG, _G * _DFE), jnp.float32),
            pltpu.VMEM((_G * _DFE, _G * _DRH), jnp.float32),
            pltpu.VMEM((_GH, _G), jnp.float32),
            pltpu.VMEM((_G, _GH), jnp.float32),
        ],
    )(x2, f2, lng2, lnb2, Wr1, wg2, We1, wr2r, bgf, br1f, be1f, br2f,
      we2c, be2)
    return out.reshape(_B, _S, _D)
